# Initial kernel scaffold; baseline (speedup 1.0000x reference)
import jax
import jax.numpy as jnp
from jax.experimental import pallas as pl


def kernel(wall_batch, x, tar_scores, geo, init_W1, init_b1, init_W2, init_b2, wall_W1, wall_b1, wall_W2, wall_b2, emb_table, emb_W, emb_b, geo_W1, geo_b1, geo_W2, geo_b2, c1_W1, c1_b1, c1_W2, c1_b2, c2_W1, c2_b1, c2_W2, c2_b2, tail_W1, tail_b1, tail_W2, tail_b2, category, batch, edge_index):
    raise NotImplementedError("write your pallas kernel here")


# speedup vs baseline: 1.1602x; 1.1602x over previous
import functools

import jax
import jax.numpy as jnp
from jax import lax
from jax.experimental import pallas as pl
from jax.experimental.pallas import tpu as pltpu
from jax.experimental.pallas import tpu_sc as plsc

N = 10000
NP = 10240
E = 160000
EP = 163840
HID = 128
EMB = 64
COND = 3 * EMB
H2 = HID + COND
BN = 2048
BE = 4096
NW = 32
EPW = EP // NW
CH_B = 256
EQ = 2
EPH = EP // EQ
CH_D = 2560
PAIRS = CH_D // 2
UNROLL = 4
NEG = -3.0e38
F32 = jnp.float32

_dn_t = (((0,), (1,)), ((), ()))
_dn_0 = (((0,), (0,)), ((), ()))


def _dot_t(a, b):
    return lax.dot_general(a, b, _dn_t, preferred_element_type=F32)


def _dot_0(a, b):
    return lax.dot_general(a, b, _dn_0, preferred_element_type=F32)


def _node_prep_body(xt, cat, bat, geo, wb,
                    iW1, ib1, iW2c, ib2c,
                    wW1, wb1, wW2, wb2,
                    tab, eW, eb,
                    gW1, gb1, gW2c, gb2c,
                    W1d, W1s, b1r,
                    condT_ref, P_ref, Q_ref):
    bn = xt.shape[0] if hasattr(xt, "shape") else BN
    tab10 = jnp.tanh(jnp.tanh(tab[...]) @ eW[...] + eb[...])
    oh_c = (cat[...] == lax.broadcasted_iota(jnp.int32, (BN, 10), 1)).astype(F32)
    classT = _dot_t(tab10, oh_c)
    wtab = jnp.tanh(jnp.tanh(wb[...] @ wW1[...] + wb1[...]) @ wW2[...] + wb2[...])
    oh_b = (bat[...] == lax.broadcasted_iota(jnp.int32, (BN, 64), 1)).astype(F32)
    wallT = _dot_t(wtab, oh_b)
    g1 = jnp.tanh(geo[...] @ gW1[...] + gb1[...])
    geoT = jnp.tanh(_dot_t(gW2c[...], g1) + gb2c[...])
    a1 = jnp.tanh(xt[...] @ iW1[...] + ib1[...])
    h0T = jnp.tanh(_dot_t(iW2c[...], a1) + ib2c[...])
    condT = jnp.concatenate([classT, wallT, geoT], axis=0)
    hT = jnp.concatenate([h0T, condT], axis=0)
    condT_ref[...] = condT
    P_ref[...] = _dot_0(hT, W1d[...]) + b1r[...]
    Q_ref[...] = _dot_0(hT, W1s[...])


_node_prep = pl.pallas_call(
    _node_prep_body,
    grid=(NP // BN,),
    in_specs=[
        pl.BlockSpec((BN, 7), lambda i: (i, 0)),
        pl.BlockSpec((BN, 1), lambda i: (i, 0)),
        pl.BlockSpec((BN, 1), lambda i: (i, 0)),
        pl.BlockSpec((BN, 2), lambda i: (i, 0)),
        pl.BlockSpec((64, 1), lambda i: (0, 0)),
        pl.BlockSpec((7, HID), lambda i: (0, 0)),
        pl.BlockSpec((1, HID), lambda i: (0, 0)),
        pl.BlockSpec((HID, HID), lambda i: (0, 0)),
        pl.BlockSpec((HID, 1), lambda i: (0, 0)),
        pl.BlockSpec((1, EMB), lambda i: (0, 0)),
        pl.BlockSpec((1, EMB), lambda i: (0, 0)),
        pl.BlockSpec((EMB, EMB), lambda i: (0, 0)),
        pl.BlockSpec((1, EMB), lambda i: (0, 0)),
        pl.BlockSpec((10, EMB), lambda i: (0, 0)),
        pl.BlockSpec((EMB, EMB), lambda i: (0, 0)),
        pl.BlockSpec((1, EMB), lambda i: (0, 0)),
        pl.BlockSpec((2, EMB), lambda i: (0, 0)),
        pl.BlockSpec((1, EMB), lambda i: (0, 0)),
        pl.BlockSpec((EMB, EMB), lambda i: (0, 0)),
        pl.BlockSpec((EMB, 1), lambda i: (0, 0)),
        pl.BlockSpec((H2, HID), lambda i: (0, 0)),
        pl.BlockSpec((H2, HID), lambda i: (0, 0)),
        pl.BlockSpec((1, HID), lambda i: (0, 0)),
    ],
    out_specs=[
        pl.BlockSpec((COND, BN), lambda i: (0, i)),
        pl.BlockSpec((BN, HID), lambda i: (i, 0)),
        pl.BlockSpec((BN, HID), lambda i: (i, 0)),
    ],
    out_shape=[
        jax.ShapeDtypeStruct((COND, NP), F32),
        jax.ShapeDtypeStruct((NP, HID), F32),
        jax.ShapeDtypeStruct((NP, HID), F32),
    ],
)


_sc_mesh = plsc.VectorSubcoreMesh(core_axis_name="c", subcore_axis_name="s")


@functools.partial(
    pl.kernel,
    out_type=(jax.ShapeDtypeStruct((EP, HID), F32),
              jax.ShapeDtypeStruct((EP, HID), F32)),
    mesh=_sc_mesh,
    scratch_types=[
        pltpu.VMEM((CH_B,), jnp.int32),
        pltpu.VMEM((CH_B,), jnp.int32),
        pltpu.VMEM((CH_B, HID), F32),
        pltpu.VMEM((CH_B, HID), F32),
        pltpu.SemaphoreType.DMA,
        pltpu.SemaphoreType.DMA,
    ],
)
def _edge_gather(P_hbm, Q_hbm, dst_hbm, src_hbm, preD_hbm, preS_hbm,
                 dsti, srci, bufP, bufQ, semP, semQ):
    wid = lax.axis_index("s") * 2 + lax.axis_index("c")
    base = wid * EPW

    def step(i, carry):
        off = base + i * CH_B
        pltpu.sync_copy(dst_hbm.at[pl.ds(off, CH_B)], dsti)
        pltpu.sync_copy(src_hbm.at[pl.ds(off, CH_B)], srci)
        cp = pltpu.async_copy(P_hbm.at[dsti], bufP, semP)
        cq = pltpu.async_copy(Q_hbm.at[srci], bufQ, semQ)
        cp.wait()
        cq.wait()
        pltpu.sync_copy(bufP, preD_hbm.at[pl.ds(off, CH_B)])
        pltpu.sync_copy(bufQ, preS_hbm.at[pl.ds(off, CH_B)])
        return carry

    lax.fori_loop(0, EPW // CH_B, step, 0)


def _edge_mlp_body(pD, pS, W2, b2c, out_ref):
    t = jnp.tanh(pD[...] + pS[...])
    out_ref[...] = _dot_t(W2[...], t) + b2c[...]


_edge_mlp = pl.pallas_call(
    _edge_mlp_body,
    grid=(EP // BE,),
    in_specs=[
        pl.BlockSpec((BE, HID), lambda i: (i, 0)),
        pl.BlockSpec((BE, HID), lambda i: (i, 0)),
        pl.BlockSpec((HID, HID), lambda i: (0, 0)),
        pl.BlockSpec((HID, 1), lambda i: (0, 0)),
    ],
    out_specs=pl.BlockSpec((HID, BE), lambda i: (0, i)),
    out_shape=jax.ShapeDtypeStruct((HID, EP), F32),
)


@functools.partial(
    pl.kernel,
    out_type=jax.ShapeDtypeStruct((EQ, HID, NP), F32),
    mesh=_sc_mesh,
    compiler_params=pltpu.CompilerParams(needs_layout_passes=False),
    scratch_types=[
        pltpu.VMEM((CH_D,), jnp.int32),
        pltpu.VMEM((8, CH_D), F32),
        pltpu.VMEM((8, NP), F32),
    ],
)
def _seg_max(MT_hbm, dst_hbm, out_hbm, dstv, mbuf, acc):
    wid = lax.axis_index("s") * 2 + lax.axis_index("c")
    cg = wid % 16
    eq = wid // 16
    lane = lax.iota(jnp.int32, 16)
    rowp = lane % 8
    colp = lane // 8
    negv = jnp.full((16,), NEG, F32)

    for r in range(8):
        def initrow(c, carry, r=r):
            acc[r, pl.ds(c * 16, 16)] = negv
            return carry
        lax.fori_loop(0, NP // 16, initrow, 0)

    def chunk_step(ci, carry):
        e0 = eq * EPH + ci * CH_D
        pltpu.sync_copy(dst_hbm.at[pl.ds(e0, CH_D)], dstv)
        pltpu.sync_copy(MT_hbm.at[pl.ds(cg * 8, 8), pl.ds(e0, CH_D)], mbuf)

        def pair_step(k, c2):
            j0 = k * (2 * UNROLL)
            for u in range(UNROLL):
                j = j0 + 2 * u
                dcol = plsc.load_gather(dstv, [j + colp])
                dsw = plsc.load_gather(dstv, [j + 1 - colp])
                mv = plsc.load_gather(mbuf, [rowp, j + colp])
                msw = plsc.load_gather(mbuf, [rowp, j + 1 - colp])
                mv = jnp.where(dcol == dsw, jnp.maximum(mv, msw), mv)
                cur = plsc.load_gather(acc, [rowp, dcol])
                plsc.store_scatter(acc, [rowp, dcol], jnp.maximum(cur, mv))
            return c2

        lax.fori_loop(0, PAIRS // UNROLL, pair_step, 0)
        return carry

    lax.fori_loop(0, EPH // CH_D, chunk_step, 0)
    pltpu.sync_copy(acc, out_hbm.at[eq, pl.ds(cg * 8, 8), :])


def _merge_h(s_blk):
    sm = jnp.max(s_blk, axis=0)
    sm = jnp.where(sm < -1.0e38, 0.0, sm)
    return jnp.tanh(sm)


def _mid_body(s, condT, W1d, W1s, b1r, P_ref, Q_ref):
    hT = jnp.concatenate([_merge_h(s[...]), condT[...]], axis=0)
    P_ref[...] = _dot_0(hT, W1d[...]) + b1r[...]
    Q_ref[...] = _dot_0(hT, W1s[...])


_mid = pl.pallas_call(
    _mid_body,
    grid=(NP // BN,),
    in_specs=[
        pl.BlockSpec((EQ, HID, BN), lambda i: (0, 0, i)),
        pl.BlockSpec((COND, BN), lambda i: (0, i)),
        pl.BlockSpec((H2, HID), lambda i: (0, 0)),
        pl.BlockSpec((H2, HID), lambda i: (0, 0)),
        pl.BlockSpec((1, HID), lambda i: (0, 0)),
    ],
    out_specs=[
        pl.BlockSpec((BN, HID), lambda i: (i, 0)),
        pl.BlockSpec((BN, HID), lambda i: (i, 0)),
    ],
    out_shape=[
        jax.ShapeDtypeStruct((NP, HID), F32),
        jax.ShapeDtypeStruct((NP, HID), F32),
    ],
)


def _tail_body(s, condT, tW1, tb1, tW2, tb2, mu_ref, std_ref):
    hT = jnp.concatenate([_merge_h(s[...]), condT[...]], axis=0)
    t = jnp.tanh(_dot_0(hT, tW1[...]) + tb1[...])
    o = t @ tW2[...] + tb2[...]
    mu_ref[...] = jnp.tanh(o[:, 0:3])
    ls = jnp.tanh(o[:, 3:6])
    std_ref[...] = jnp.exp(-5.0 + 3.5 * (ls + 1.0))


_tail = pl.pallas_call(
    _tail_body,
    grid=(NP // BN,),
    in_specs=[
        pl.BlockSpec((EQ, HID, BN), lambda i: (0, 0, i)),
        pl.BlockSpec((COND, BN), lambda i: (0, i)),
        pl.BlockSpec((H2, HID), lambda i: (0, 0)),
        pl.BlockSpec((1, HID), lambda i: (0, 0)),
        pl.BlockSpec((HID, 6), lambda i: (0, 0)),
        pl.BlockSpec((1, 6), lambda i: (0, 0)),
    ],
    out_specs=[
        pl.BlockSpec((BN, 3), lambda i: (i, 0)),
        pl.BlockSpec((BN, 3), lambda i: (i, 0)),
    ],
    out_shape=[
        jax.ShapeDtypeStruct((NP, 3), F32),
        jax.ShapeDtypeStruct((NP, 3), F32),
    ],
)


def kernel(wall_batch, x, tar_scores, geo,
           init_W1, init_b1, init_W2, init_b2,
           wall_W1, wall_b1, wall_W2, wall_b2,
           emb_table, emb_W, emb_b,
           geo_W1, geo_b1, geo_W2, geo_b2,
           c1_W1, c1_b1, c1_W2, c1_b2,
           c2_W1, c2_b1, c2_W2, c2_b2,
           tail_W1, tail_b1, tail_W2, tail_b2,
           category, batch, edge_index):
    zn = NP - N
    xt = jnp.concatenate([x, tar_scores], axis=1)
    xt = jnp.concatenate([xt, jnp.zeros((zn, 7), F32)], axis=0)
    cat_p = jnp.concatenate([category, jnp.zeros((zn, 1), jnp.int32)], axis=0)
    bat2 = jnp.concatenate([batch.reshape(N, 1), jnp.zeros((zn, 1), jnp.int32)], axis=0)
    geo_p = jnp.concatenate([geo, jnp.zeros((zn, 2), F32)], axis=0)
    pad_d = jnp.full((EP - E,), N, jnp.int32)
    pad_s = jnp.zeros((EP - E,), jnp.int32)
    dst = jnp.concatenate([edge_index[1], pad_d])
    src = jnp.concatenate([edge_index[0], pad_s])
    W1d1, W1s1 = c1_W1[:H2] - c1_W1[H2:], c1_W1[H2:]
    W1d2, W1s2 = c2_W1[:H2] - c2_W1[H2:], c2_W1[H2:]

    condT, P1, Q1 = _node_prep(
        xt, cat_p, bat2, geo_p, wall_batch,
        init_W1, init_b1.reshape(1, HID), init_W2, init_b2.reshape(HID, 1),
        wall_W1.reshape(1, EMB), wall_b1.reshape(1, EMB),
        wall_W2, wall_b2.reshape(1, EMB),
        emb_table, emb_W, emb_b.reshape(1, EMB),
        geo_W1, geo_b1.reshape(1, EMB), geo_W2, geo_b2.reshape(EMB, 1),
        W1d1, W1s1, c1_b1.reshape(1, HID))

    preD, preS = _edge_gather(P1, Q1, dst, src)
    MT1 = _edge_mlp(preD, preS, c1_W2, c1_b2.reshape(HID, 1))
    S1 = _seg_max(MT1, dst)

    P2, Q2 = _mid(S1, condT, W1d2, W1s2, c2_b1.reshape(1, HID))
    preD2, preS2 = _edge_gather(P2, Q2, dst, src)
    MT2 = _edge_mlp(preD2, preS2, c2_W2, c2_b2.reshape(HID, 1))
    S2 = _seg_max(MT2, dst)

    mu, std = _tail(S2, condT, tail_W1, tail_b1.reshape(1, HID),
                    tail_W2, tail_b2.reshape(1, 6))
    return (mu[:N], std[:N])


# speedup vs baseline: 1.4013x; 1.2078x over previous
import functools

import jax
import jax.numpy as jnp
from jax import lax
from jax.experimental import pallas as pl
from jax.experimental.pallas import tpu as pltpu
from jax.experimental.pallas import tpu_sc as plsc

N = 10000
NP = 10240
E = 160000
EP = 163840
HID = 128
EMB = 64
COND = 3 * EMB
H2 = HID + COND
BN = 2048
BE = 4096
NW = 32
EPW = EP // NW
CH_B = 256
EQ = 2
EPH = EP // EQ
CH_D = 2560
PAIRS = CH_D // 2
UNROLL = 4
NEG = -3.0e38
F32 = jnp.float32

_dn_t = (((0,), (1,)), ((), ()))
_dn_0 = (((0,), (0,)), ((), ()))


def _dot_t(a, b):
    return lax.dot_general(a, b, _dn_t, preferred_element_type=F32)


def _dot_0(a, b):
    return lax.dot_general(a, b, _dn_0, preferred_element_type=F32)


def _node_prep_body(xt, cat, bat, geo, wb,
                    iW1, ib1, iW2c, ib2c,
                    wW1, wb1, wW2, wb2,
                    tab, eW, eb,
                    gW1, gb1, gW2c, gb2c,
                    W1d, W1s, b1r,
                    condT_ref, P_ref, Q_ref):
    bn = xt.shape[0] if hasattr(xt, "shape") else BN
    tab10 = jnp.tanh(jnp.tanh(tab[...]) @ eW[...] + eb[...])
    oh_c = (cat[...] == lax.broadcasted_iota(jnp.int32, (BN, 10), 1)).astype(F32)
    classT = _dot_t(tab10, oh_c)
    wtab = jnp.tanh(jnp.tanh(wb[...] @ wW1[...] + wb1[...]) @ wW2[...] + wb2[...])
    oh_b = (bat[...] == lax.broadcasted_iota(jnp.int32, (BN, 64), 1)).astype(F32)
    wallT = _dot_t(wtab, oh_b)
    g1 = jnp.tanh(geo[...] @ gW1[...] + gb1[...])
    geoT = jnp.tanh(_dot_t(gW2c[...], g1) + gb2c[...])
    a1 = jnp.tanh(xt[...] @ iW1[...] + ib1[...])
    h0T = jnp.tanh(_dot_t(iW2c[...], a1) + ib2c[...])
    condT = jnp.concatenate([classT, wallT, geoT], axis=0)
    hT = jnp.concatenate([h0T, condT], axis=0)
    condT_ref[...] = condT
    P_ref[...] = _dot_0(hT, W1d[...]) + b1r[...]
    Q_ref[...] = _dot_0(hT, W1s[...])


_node_prep = pl.pallas_call(
    _node_prep_body,
    grid=(NP // BN,),
    in_specs=[
        pl.BlockSpec((BN, 7), lambda i: (i, 0)),
        pl.BlockSpec((BN, 1), lambda i: (i, 0)),
        pl.BlockSpec((BN, 1), lambda i: (i, 0)),
        pl.BlockSpec((BN, 2), lambda i: (i, 0)),
        pl.BlockSpec((64, 1), lambda i: (0, 0)),
        pl.BlockSpec((7, HID), lambda i: (0, 0)),
        pl.BlockSpec((1, HID), lambda i: (0, 0)),
        pl.BlockSpec((HID, HID), lambda i: (0, 0)),
        pl.BlockSpec((HID, 1), lambda i: (0, 0)),
        pl.BlockSpec((1, EMB), lambda i: (0, 0)),
        pl.BlockSpec((1, EMB), lambda i: (0, 0)),
        pl.BlockSpec((EMB, EMB), lambda i: (0, 0)),
        pl.BlockSpec((1, EMB), lambda i: (0, 0)),
        pl.BlockSpec((10, EMB), lambda i: (0, 0)),
        pl.BlockSpec((EMB, EMB), lambda i: (0, 0)),
        pl.BlockSpec((1, EMB), lambda i: (0, 0)),
        pl.BlockSpec((2, EMB), lambda i: (0, 0)),
        pl.BlockSpec((1, EMB), lambda i: (0, 0)),
        pl.BlockSpec((EMB, EMB), lambda i: (0, 0)),
        pl.BlockSpec((EMB, 1), lambda i: (0, 0)),
        pl.BlockSpec((H2, HID), lambda i: (0, 0)),
        pl.BlockSpec((H2, HID), lambda i: (0, 0)),
        pl.BlockSpec((1, HID), lambda i: (0, 0)),
    ],
    out_specs=[
        pl.BlockSpec((COND, BN), lambda i: (0, i)),
        pl.BlockSpec((BN, HID), lambda i: (i, 0)),
        pl.BlockSpec((BN, HID), lambda i: (i, 0)),
    ],
    out_shape=[
        jax.ShapeDtypeStruct((COND, NP), F32),
        jax.ShapeDtypeStruct((NP, HID), F32),
        jax.ShapeDtypeStruct((NP, HID), F32),
    ],
)


_sc_mesh = plsc.VectorSubcoreMesh(core_axis_name="c", subcore_axis_name="s")


@functools.partial(
    pl.kernel,
    out_type=(jax.ShapeDtypeStruct((EP, HID), F32),
              jax.ShapeDtypeStruct((EP, HID), F32)),
    mesh=_sc_mesh,
    scratch_types=[
        pltpu.VMEM((CH_B,), jnp.int32),
        pltpu.VMEM((CH_B,), jnp.int32),
        pltpu.VMEM((CH_B, HID), F32),
        pltpu.VMEM((CH_B, HID), F32),
        pltpu.SemaphoreType.DMA,
        pltpu.SemaphoreType.DMA,
    ],
)
def _edge_gather(P_hbm, Q_hbm, dst_hbm, src_hbm, preD_hbm, preS_hbm,
                 dsti, srci, bufP, bufQ, semP, semQ):
    wid = lax.axis_index("s") * 2 + lax.axis_index("c")
    base = wid * EPW

    def step(i, carry):
        off = base + i * CH_B
        pltpu.sync_copy(dst_hbm.at[pl.ds(off, CH_B)], dsti)
        pltpu.sync_copy(src_hbm.at[pl.ds(off, CH_B)], srci)
        cp = pltpu.async_copy(P_hbm.at[dsti], bufP, semP)
        cq = pltpu.async_copy(Q_hbm.at[srci], bufQ, semQ)
        cp.wait()
        cq.wait()
        pltpu.sync_copy(bufP, preD_hbm.at[pl.ds(off, CH_B)])
        pltpu.sync_copy(bufQ, preS_hbm.at[pl.ds(off, CH_B)])
        return carry

    lax.fori_loop(0, EPW // CH_B, step, 0)


def _edge_mlp_body(pD, pS, dstp, W2, b2r, eye, out_ref):
    t = jnp.tanh(pD[...] + pS[...])
    M = t @ W2[...] + b2r[...]
    Mr = M.reshape(BE // 2, 2, HID)
    deq = dstp[:, 0:1] == dstp[:, 1:2]
    mm = jnp.max(Mr, axis=1)
    me = jnp.where(deq, mm, Mr[:, 0, :])
    mo = jnp.where(deq, mm, Mr[:, 1, :])
    M2 = jnp.concatenate([me[:, None, :], mo[:, None, :]], axis=1).reshape(BE, HID)
    out_ref[...] = _dot_t(eye[...], M2)


_edge_mlp = pl.pallas_call(
    _edge_mlp_body,
    grid=(EP // BE,),
    in_specs=[
        pl.BlockSpec((BE, HID), lambda i: (i, 0)),
        pl.BlockSpec((BE, HID), lambda i: (i, 0)),
        pl.BlockSpec((BE // 2, 2), lambda i: (i, 0)),
        pl.BlockSpec((HID, HID), lambda i: (0, 0)),
        pl.BlockSpec((1, HID), lambda i: (0, 0)),
        pl.BlockSpec((HID, HID), lambda i: (0, 0)),
    ],
    out_specs=pl.BlockSpec((HID, BE), lambda i: (0, i)),
    out_shape=jax.ShapeDtypeStruct((HID, EP), F32),
)


@functools.partial(
    pl.kernel,
    out_type=jax.ShapeDtypeStruct((EQ, HID, NP), F32),
    mesh=_sc_mesh,
    compiler_params=pltpu.CompilerParams(needs_layout_passes=False),
    scratch_types=[
        pltpu.VMEM((CH_D,), jnp.int32),
        pltpu.VMEM((8, CH_D), F32),
        pltpu.VMEM((8, NP), F32),
    ],
)
def _seg_max(MT_hbm, dst_hbm, out_hbm, dstv, mbuf, acc):
    wid = lax.axis_index("s") * 2 + lax.axis_index("c")
    cg = wid % 16
    eq = wid // 16
    lane = lax.iota(jnp.int32, 16)
    rowp = lane % 8
    colp = lane // 8
    negv = jnp.full((16,), NEG, F32)

    for r in range(8):
        def initrow(c, carry, r=r):
            acc[r, pl.ds(c * 16, 16)] = negv
            return carry
        lax.fori_loop(0, NP // 16, initrow, 0)

    def chunk_step(ci, carry):
        e0 = eq * EPH + ci * CH_D
        pltpu.sync_copy(dst_hbm.at[pl.ds(e0, CH_D)], dstv)
        pltpu.sync_copy(MT_hbm.at[pl.ds(cg * 8, 8), pl.ds(e0, CH_D)], mbuf)

        def pair_step(k, c2):
            j0 = k * (2 * UNROLL)
            dcols = []
            mvs = []
            for u in range(UNROLL):
                j = j0 + 2 * u
                dcols.append(plsc.load_gather(dstv, [j + colp]))
                mvs.append(plsc.load_gather(mbuf, [rowp, j + colp]))
            for u in range(UNROLL):
                cur = plsc.load_gather(acc, [rowp, dcols[u]])
                plsc.store_scatter(acc, [rowp, dcols[u]], jnp.maximum(cur, mvs[u]))
            return c2

        lax.fori_loop(0, PAIRS // UNROLL, pair_step, 0)
        return carry

    lax.fori_loop(0, EPH // CH_D, chunk_step, 0)
    pltpu.sync_copy(acc, out_hbm.at[eq, pl.ds(cg * 8, 8), :])


def _merge_h(s_blk):
    sm = jnp.max(s_blk, axis=0)
    sm = jnp.where(sm < -1.0e38, 0.0, sm)
    return jnp.tanh(sm)


def _mid_body(s, condT, W1d, W1s, b1r, P_ref, Q_ref):
    hT = jnp.concatenate([_merge_h(s[...]), condT[...]], axis=0)
    P_ref[...] = _dot_0(hT, W1d[...]) + b1r[...]
    Q_ref[...] = _dot_0(hT, W1s[...])


_mid = pl.pallas_call(
    _mid_body,
    grid=(NP // BN,),
    in_specs=[
        pl.BlockSpec((EQ, HID, BN), lambda i: (0, 0, i)),
        pl.BlockSpec((COND, BN), lambda i: (0, i)),
        pl.BlockSpec((H2, HID), lambda i: (0, 0)),
        pl.BlockSpec((H2, HID), lambda i: (0, 0)),
        pl.BlockSpec((1, HID), lambda i: (0, 0)),
    ],
    out_specs=[
        pl.BlockSpec((BN, HID), lambda i: (i, 0)),
        pl.BlockSpec((BN, HID), lambda i: (i, 0)),
    ],
    out_shape=[
        jax.ShapeDtypeStruct((NP, HID), F32),
        jax.ShapeDtypeStruct((NP, HID), F32),
    ],
)


def _tail_body(s, condT, tW1, tb1, tW2, tb2, mu_ref, std_ref):
    hT = jnp.concatenate([_merge_h(s[...]), condT[...]], axis=0)
    t = jnp.tanh(_dot_0(hT, tW1[...]) + tb1[...])
    o = t @ tW2[...] + tb2[...]
    mu_ref[...] = jnp.tanh(o[:, 0:3])
    ls = jnp.tanh(o[:, 3:6])
    std_ref[...] = jnp.exp(-5.0 + 3.5 * (ls + 1.0))


_tail = pl.pallas_call(
    _tail_body,
    grid=(NP // BN,),
    in_specs=[
        pl.BlockSpec((EQ, HID, BN), lambda i: (0, 0, i)),
        pl.BlockSpec((COND, BN), lambda i: (0, i)),
        pl.BlockSpec((H2, HID), lambda i: (0, 0)),
        pl.BlockSpec((1, HID), lambda i: (0, 0)),
        pl.BlockSpec((HID, 6), lambda i: (0, 0)),
        pl.BlockSpec((1, 6), lambda i: (0, 0)),
    ],
    out_specs=[
        pl.BlockSpec((BN, 3), lambda i: (i, 0)),
        pl.BlockSpec((BN, 3), lambda i: (i, 0)),
    ],
    out_shape=[
        jax.ShapeDtypeStruct((NP, 3), F32),
        jax.ShapeDtypeStruct((NP, 3), F32),
    ],
)


def kernel(wall_batch, x, tar_scores, geo,
           init_W1, init_b1, init_W2, init_b2,
           wall_W1, wall_b1, wall_W2, wall_b2,
           emb_table, emb_W, emb_b,
           geo_W1, geo_b1, geo_W2, geo_b2,
           c1_W1, c1_b1, c1_W2, c1_b2,
           c2_W1, c2_b1, c2_W2, c2_b2,
           tail_W1, tail_b1, tail_W2, tail_b2,
           category, batch, edge_index):
    zn = NP - N
    xt = jnp.concatenate([x, tar_scores], axis=1)
    xt = jnp.concatenate([xt, jnp.zeros((zn, 7), F32)], axis=0)
    cat_p = jnp.concatenate([category, jnp.zeros((zn, 1), jnp.int32)], axis=0)
    bat2 = jnp.concatenate([batch.reshape(N, 1), jnp.zeros((zn, 1), jnp.int32)], axis=0)
    geo_p = jnp.concatenate([geo, jnp.zeros((zn, 2), F32)], axis=0)
    pad_d = jnp.full((EP - E,), N, jnp.int32)
    pad_s = jnp.zeros((EP - E,), jnp.int32)
    dst = jnp.concatenate([edge_index[1], pad_d])
    src = jnp.concatenate([edge_index[0], pad_s])
    W1d1, W1s1 = c1_W1[:H2] - c1_W1[H2:], c1_W1[H2:]
    W1d2, W1s2 = c2_W1[:H2] - c2_W1[H2:], c2_W1[H2:]

    condT, P1, Q1 = _node_prep(
        xt, cat_p, bat2, geo_p, wall_batch,
        init_W1, init_b1.reshape(1, HID), init_W2, init_b2.reshape(HID, 1),
        wall_W1.reshape(1, EMB), wall_b1.reshape(1, EMB),
        wall_W2, wall_b2.reshape(1, EMB),
        emb_table, emb_W, emb_b.reshape(1, EMB),
        geo_W1, geo_b1.reshape(1, EMB), geo_W2, geo_b2.reshape(EMB, 1),
        W1d1, W1s1, c1_b1.reshape(1, HID))

    dstp = dst.reshape(EP // 2, 2)
    eye = jnp.eye(HID, dtype=F32)
    preD, preS = _edge_gather(P1, Q1, dst, src)
    MT1 = _edge_mlp(preD, preS, dstp, c1_W2, c1_b2.reshape(1, HID), eye)
    S1 = _seg_max(MT1, dst)

    P2, Q2 = _mid(S1, condT, W1d2, W1s2, c2_b1.reshape(1, HID))
    preD2, preS2 = _edge_gather(P2, Q2, dst, src)
    MT2 = _edge_mlp(preD2, preS2, dstp, c2_W2, c2_b2.reshape(1, HID), eye)
    S2 = _seg_max(MT2, dst)

    mu, std = _tail(S2, condT, tail_W1, tail_b1.reshape(1, HID),
                    tail_W2, tail_b2.reshape(1, 6))
    return (mu[:N], std[:N])


# speedup vs baseline: 1.4123x; 1.0079x over previous
import functools

import jax
import jax.numpy as jnp
from jax import lax
from jax.experimental import pallas as pl
from jax.experimental.pallas import tpu as pltpu
from jax.experimental.pallas import tpu_sc as plsc

N = 10000
NP = 10240
E = 160000
EP = 163840
HID = 128
EMB = 64
COND = 3 * EMB
H2 = HID + COND
BN = 2048
BE = 4096
NW = 32
EPW = EP // NW
CH_B = 128
NCH_B = EPW // CH_B
EQ = 2
EPH = EP // EQ
CH_D = 2560
PAIRS = CH_D // 2
UNROLL = 8
NEG = -3.0e38
F32 = jnp.float32

_dn_t = (((0,), (1,)), ((), ()))
_dn_0 = (((0,), (0,)), ((), ()))


def _dot_t(a, b):
    return lax.dot_general(a, b, _dn_t, preferred_element_type=F32)


def _dot_0(a, b):
    return lax.dot_general(a, b, _dn_0, preferred_element_type=F32)


def _node_prep_body(xt, cat, bat, geo, wb,
                    iW1, ib1, iW2c, ib2c,
                    wW1, wb1, wW2, wb2,
                    tab, eW, eb,
                    gW1, gb1, gW2c, gb2c,
                    W1d, W1s, b1r,
                    condT_ref, P_ref, Q_ref):
    bn = xt.shape[0] if hasattr(xt, "shape") else BN
    tab10 = jnp.tanh(jnp.tanh(tab[...]) @ eW[...] + eb[...])
    oh_c = (cat[...] == lax.broadcasted_iota(jnp.int32, (BN, 10), 1)).astype(F32)
    classT = _dot_t(tab10, oh_c)
    wtab = jnp.tanh(jnp.tanh(wb[...] @ wW1[...] + wb1[...]) @ wW2[...] + wb2[...])
    oh_b = (bat[...] == lax.broadcasted_iota(jnp.int32, (BN, 64), 1)).astype(F32)
    wallT = _dot_t(wtab, oh_b)
    g1 = jnp.tanh(geo[...] @ gW1[...] + gb1[...])
    geoT = jnp.tanh(_dot_t(gW2c[...], g1) + gb2c[...])
    a1 = jnp.tanh(xt[...] @ iW1[...] + ib1[...])
    h0T = jnp.tanh(_dot_t(iW2c[...], a1) + ib2c[...])
    condT = jnp.concatenate([classT, wallT, geoT], axis=0)
    hT = jnp.concatenate([h0T, condT], axis=0)
    condT_ref[...] = condT
    P_ref[...] = _dot_0(hT, W1d[...]) + b1r[...]
    Q_ref[...] = _dot_0(hT, W1s[...])


_node_prep = pl.pallas_call(
    _node_prep_body,
    grid=(NP // BN,),
    in_specs=[
        pl.BlockSpec((BN, 7), lambda i: (i, 0)),
        pl.BlockSpec((BN, 1), lambda i: (i, 0)),
        pl.BlockSpec((BN, 1), lambda i: (i, 0)),
        pl.BlockSpec((BN, 2), lambda i: (i, 0)),
        pl.BlockSpec((64, 1), lambda i: (0, 0)),
        pl.BlockSpec((7, HID), lambda i: (0, 0)),
        pl.BlockSpec((1, HID), lambda i: (0, 0)),
        pl.BlockSpec((HID, HID), lambda i: (0, 0)),
        pl.BlockSpec((HID, 1), lambda i: (0, 0)),
        pl.BlockSpec((1, EMB), lambda i: (0, 0)),
        pl.BlockSpec((1, EMB), lambda i: (0, 0)),
        pl.BlockSpec((EMB, EMB), lambda i: (0, 0)),
        pl.BlockSpec((1, EMB), lambda i: (0, 0)),
        pl.BlockSpec((10, EMB), lambda i: (0, 0)),
        pl.BlockSpec((EMB, EMB), lambda i: (0, 0)),
        pl.BlockSpec((1, EMB), lambda i: (0, 0)),
        pl.BlockSpec((2, EMB), lambda i: (0, 0)),
        pl.BlockSpec((1, EMB), lambda i: (0, 0)),
        pl.BlockSpec((EMB, EMB), lambda i: (0, 0)),
        pl.BlockSpec((EMB, 1), lambda i: (0, 0)),
        pl.BlockSpec((H2, HID), lambda i: (0, 0)),
        pl.BlockSpec((H2, HID), lambda i: (0, 0)),
        pl.BlockSpec((1, HID), lambda i: (0, 0)),
    ],
    out_specs=[
        pl.BlockSpec((COND, BN), lambda i: (0, i)),
        pl.BlockSpec((BN, HID), lambda i: (i, 0)),
        pl.BlockSpec((BN, HID), lambda i: (i, 0)),
    ],
    out_shape=[
        jax.ShapeDtypeStruct((COND, NP), F32),
        jax.ShapeDtypeStruct((NP, HID), F32),
        jax.ShapeDtypeStruct((NP, HID), F32),
    ],
)


_sc_mesh = plsc.VectorSubcoreMesh(core_axis_name="c", subcore_axis_name="s")


@functools.partial(
    pl.kernel,
    out_type=(jax.ShapeDtypeStruct((EP, HID), F32),
              jax.ShapeDtypeStruct((EP, HID), F32)),
    mesh=_sc_mesh,
    scratch_types=[
        pltpu.VMEM((2, CH_B), jnp.int32),
        pltpu.VMEM((2, CH_B), jnp.int32),
        pltpu.VMEM((2, CH_B, HID), F32),
        pltpu.VMEM((2, CH_B, HID), F32),
        [pltpu.SemaphoreType.DMA] * 2,
        [pltpu.SemaphoreType.DMA] * 2,
        [pltpu.SemaphoreType.DMA] * 2,
        [pltpu.SemaphoreType.DMA] * 2,
    ],
)
def _edge_gather(P_hbm, Q_hbm, dst_hbm, src_hbm, preD_hbm, preS_hbm,
                 dsti, srci, bufP, bufQ, gP, gQ, sP, sQ):
    wid = lax.axis_index("s") * 2 + lax.axis_index("c")
    base = wid * EPW

    def issue(c, b):
        off = base + c * CH_B
        pltpu.sync_copy(dst_hbm.at[pl.ds(off, CH_B)], dsti.at[b])
        pltpu.sync_copy(src_hbm.at[pl.ds(off, CH_B)], srci.at[b])
        pltpu.async_copy(P_hbm.at[dsti.at[b]], bufP.at[b], gP[b])
        pltpu.async_copy(Q_hbm.at[srci.at[b]], bufQ.at[b], gQ[b])

    def drain(c, b):
        off = base + c * CH_B
        pltpu.make_async_copy(P_hbm.at[dsti.at[b]], bufP.at[b], gP[b]).wait()
        pltpu.make_async_copy(Q_hbm.at[srci.at[b]], bufQ.at[b], gQ[b]).wait()
        pltpu.async_copy(bufP.at[b], preD_hbm.at[pl.ds(off, CH_B)], sP[b])
        pltpu.async_copy(bufQ.at[b], preS_hbm.at[pl.ds(off, CH_B)], sQ[b])

    def wstore(c, b):
        off = base + c * CH_B
        pltpu.make_async_copy(bufP.at[b], preD_hbm.at[pl.ds(off, CH_B)], sP[b]).wait()
        pltpu.make_async_copy(bufQ.at[b], preS_hbm.at[pl.ds(off, CH_B)], sQ[b]).wait()

    for c in range(NCH_B):
        b = c % 2
        if c >= 2:
            wstore(c - 2, b)
        issue(c, b)
        if c >= 1:
            drain(c - 1, 1 - b)
    drain(NCH_B - 1, (NCH_B - 1) % 2)
    wstore(NCH_B - 2, (NCH_B - 2) % 2)
    wstore(NCH_B - 1, (NCH_B - 1) % 2)


def _edge_mlp_body(pD, pS, dstp, W2, b2r, eye, out_ref):
    t = jnp.tanh(pD[...] + pS[...])
    M = t @ W2[...] + b2r[...]
    Mr = M.reshape(BE // 2, 2, HID)
    deq = dstp[:, 0:1] == dstp[:, 1:2]
    mm = jnp.max(Mr, axis=1)
    me = jnp.where(deq, mm, Mr[:, 0, :])
    mo = jnp.where(deq, mm, Mr[:, 1, :])
    M2 = jnp.concatenate([me[:, None, :], mo[:, None, :]], axis=1).reshape(BE, HID)
    out_ref[...] = _dot_t(eye[...], M2)


_edge_mlp = pl.pallas_call(
    _edge_mlp_body,
    grid=(EP // BE,),
    in_specs=[
        pl.BlockSpec((BE, HID), lambda i: (i, 0)),
        pl.BlockSpec((BE, HID), lambda i: (i, 0)),
        pl.BlockSpec((BE // 2, 2), lambda i: (i, 0)),
        pl.BlockSpec((HID, HID), lambda i: (0, 0)),
        pl.BlockSpec((1, HID), lambda i: (0, 0)),
        pl.BlockSpec((HID, HID), lambda i: (0, 0)),
    ],
    out_specs=pl.BlockSpec((HID, BE), lambda i: (0, i)),
    out_shape=jax.ShapeDtypeStruct((HID, EP), F32),
)


@functools.partial(
    pl.kernel,
    out_type=jax.ShapeDtypeStruct((EQ, HID, NP), F32),
    mesh=_sc_mesh,
    compiler_params=pltpu.CompilerParams(needs_layout_passes=False),
    scratch_types=[
        pltpu.VMEM((CH_D,), jnp.int32),
        pltpu.VMEM((8, CH_D), F32),
        pltpu.VMEM((8, NP), F32),
    ],
)
def _seg_max(MT_hbm, dst_hbm, out_hbm, dstv, mbuf, acc):
    wid = lax.axis_index("s") * 2 + lax.axis_index("c")
    cg = wid % 16
    eq = wid // 16
    lane = lax.iota(jnp.int32, 16)
    rowp = lane % 8
    colp = lane // 8
    negv = jnp.full((16,), NEG, F32)

    for r in range(8):
        def initrow(c, carry, r=r):
            acc[r, pl.ds(c * 16, 16)] = negv
            return carry
        lax.fori_loop(0, NP // 16, initrow, 0)

    def chunk_step(ci, carry):
        e0 = eq * EPH + ci * CH_D
        pltpu.sync_copy(dst_hbm.at[pl.ds(e0, CH_D)], dstv)
        pltpu.sync_copy(MT_hbm.at[pl.ds(cg * 8, 8), pl.ds(e0, CH_D)], mbuf)

        def pair_step(k, c2):
            j0 = k * (2 * UNROLL)
            dcols = []
            mvs = []
            for u in range(UNROLL):
                j = j0 + 2 * u
                dcols.append(plsc.load_gather(dstv, [j + colp]))
                mvs.append(plsc.load_gather(mbuf, [rowp, j + colp]))
            for u in range(UNROLL):
                cur = plsc.load_gather(acc, [rowp, dcols[u]])
                plsc.store_scatter(acc, [rowp, dcols[u]], jnp.maximum(cur, mvs[u]))
            return c2

        lax.fori_loop(0, PAIRS // UNROLL, pair_step, 0)
        return carry

    lax.fori_loop(0, EPH // CH_D, chunk_step, 0)
    pltpu.sync_copy(acc, out_hbm.at[eq, pl.ds(cg * 8, 8), :])


def _merge_h(s_blk):
    sm = jnp.max(s_blk, axis=0)
    sm = jnp.where(sm < -1.0e38, 0.0, sm)
    return jnp.tanh(sm)


def _mid_body(s, condT, W1d, W1s, b1r, P_ref, Q_ref):
    hT = jnp.concatenate([_merge_h(s[...]), condT[...]], axis=0)
    P_ref[...] = _dot_0(hT, W1d[...]) + b1r[...]
    Q_ref[...] = _dot_0(hT, W1s[...])


_mid = pl.pallas_call(
    _mid_body,
    grid=(NP // BN,),
    in_specs=[
        pl.BlockSpec((EQ, HID, BN), lambda i: (0, 0, i)),
        pl.BlockSpec((COND, BN), lambda i: (0, i)),
        pl.BlockSpec((H2, HID), lambda i: (0, 0)),
        pl.BlockSpec((H2, HID), lambda i: (0, 0)),
        pl.BlockSpec((1, HID), lambda i: (0, 0)),
    ],
    out_specs=[
        pl.BlockSpec((BN, HID), lambda i: (i, 0)),
        pl.BlockSpec((BN, HID), lambda i: (i, 0)),
    ],
    out_shape=[
        jax.ShapeDtypeStruct((NP, HID), F32),
        jax.ShapeDtypeStruct((NP, HID), F32),
    ],
)


def _tail_body(s, condT, tW1, tb1, tW2, tb2, mu_ref, std_ref):
    hT = jnp.concatenate([_merge_h(s[...]), condT[...]], axis=0)
    t = jnp.tanh(_dot_0(hT, tW1[...]) + tb1[...])
    o = t @ tW2[...] + tb2[...]
    mu_ref[...] = jnp.tanh(o[:, 0:3])
    ls = jnp.tanh(o[:, 3:6])
    std_ref[...] = jnp.exp(-5.0 + 3.5 * (ls + 1.0))


_tail = pl.pallas_call(
    _tail_body,
    grid=(NP // BN,),
    in_specs=[
        pl.BlockSpec((EQ, HID, BN), lambda i: (0, 0, i)),
        pl.BlockSpec((COND, BN), lambda i: (0, i)),
        pl.BlockSpec((H2, HID), lambda i: (0, 0)),
        pl.BlockSpec((1, HID), lambda i: (0, 0)),
        pl.BlockSpec((HID, 6), lambda i: (0, 0)),
        pl.BlockSpec((1, 6), lambda i: (0, 0)),
    ],
    out_specs=[
        pl.BlockSpec((BN, 3), lambda i: (i, 0)),
        pl.BlockSpec((BN, 3), lambda i: (i, 0)),
    ],
    out_shape=[
        jax.ShapeDtypeStruct((NP, 3), F32),
        jax.ShapeDtypeStruct((NP, 3), F32),
    ],
)


def kernel(wall_batch, x, tar_scores, geo,
           init_W1, init_b1, init_W2, init_b2,
           wall_W1, wall_b1, wall_W2, wall_b2,
           emb_table, emb_W, emb_b,
           geo_W1, geo_b1, geo_W2, geo_b2,
           c1_W1, c1_b1, c1_W2, c1_b2,
           c2_W1, c2_b1, c2_W2, c2_b2,
           tail_W1, tail_b1, tail_W2, tail_b2,
           category, batch, edge_index):
    zn = NP - N
    xt = jnp.concatenate([x, tar_scores], axis=1)
    xt = jnp.concatenate([xt, jnp.zeros((zn, 7), F32)], axis=0)
    cat_p = jnp.concatenate([category, jnp.zeros((zn, 1), jnp.int32)], axis=0)
    bat2 = jnp.concatenate([batch.reshape(N, 1), jnp.zeros((zn, 1), jnp.int32)], axis=0)
    geo_p = jnp.concatenate([geo, jnp.zeros((zn, 2), F32)], axis=0)
    pad_d = jnp.full((EP - E,), N, jnp.int32)
    pad_s = jnp.zeros((EP - E,), jnp.int32)
    dst = jnp.concatenate([edge_index[1], pad_d])
    src = jnp.concatenate([edge_index[0], pad_s])
    W1d1, W1s1 = c1_W1[:H2] - c1_W1[H2:], c1_W1[H2:]
    W1d2, W1s2 = c2_W1[:H2] - c2_W1[H2:], c2_W1[H2:]

    condT, P1, Q1 = _node_prep(
        xt, cat_p, bat2, geo_p, wall_batch,
        init_W1, init_b1.reshape(1, HID), init_W2, init_b2.reshape(HID, 1),
        wall_W1.reshape(1, EMB), wall_b1.reshape(1, EMB),
        wall_W2, wall_b2.reshape(1, EMB),
        emb_table, emb_W, emb_b.reshape(1, EMB),
        geo_W1, geo_b1.reshape(1, EMB), geo_W2, geo_b2.reshape(EMB, 1),
        W1d1, W1s1, c1_b1.reshape(1, HID))

    dstp = dst.reshape(EP // 2, 2)
    eye = jnp.eye(HID, dtype=F32)
    preD, preS = _edge_gather(P1, Q1, dst, src)
    MT1 = _edge_mlp(preD, preS, dstp, c1_W2, c1_b2.reshape(1, HID), eye)
    S1 = _seg_max(MT1, dst)

    P2, Q2 = _mid(S1, condT, W1d2, W1s2, c2_b1.reshape(1, HID))
    preD2, preS2 = _edge_gather(P2, Q2, dst, src)
    MT2 = _edge_mlp(preD2, preS2, dstp, c2_W2, c2_b2.reshape(1, HID), eye)
    S2 = _seg_max(MT2, dst)

    mu, std = _tail(S2, condT, tail_W1, tail_b1.reshape(1, HID),
                    tail_W2, tail_b2.reshape(1, 6))
    return (mu[:N], std[:N])


# speedup vs baseline: 1.4865x; 1.0525x over previous
import functools

import jax
import jax.numpy as jnp
from jax import lax
from jax.experimental import pallas as pl
from jax.experimental.pallas import tpu as pltpu
from jax.experimental.pallas import tpu_sc as plsc

N = 10000
NP = 10240
E = 160000
EP = 163840
HID = 128
EMB = 64
COND = 3 * EMB
H2 = HID + COND
BN = 2048
BE = 4096
NW = 32
EPW = EP // NW
CH_B = 128
NCH_B = EPW // CH_B
EQ = 2
EPH = EP // EQ
CH_D = 5120
PAIRS = CH_D // 2
UNROLL = 4
NEG = -3.0e38
F32 = jnp.float32

_dn_t = (((0,), (1,)), ((), ()))
_dn_0 = (((0,), (0,)), ((), ()))


def _dot_t(a, b):
    return lax.dot_general(a, b, _dn_t, preferred_element_type=F32)


def _dot_0(a, b):
    return lax.dot_general(a, b, _dn_0, preferred_element_type=F32)


def _node_prep_body(xt, cat, bat, geo, wb,
                    iW1, ib1, iW2c, ib2c,
                    wW1, wb1, wW2, wb2,
                    tab, eW, eb,
                    gW1, gb1, gW2c, gb2c,
                    W1d, W1s, b1r,
                    condT_ref, P_ref, Q_ref):
    bn = xt.shape[0] if hasattr(xt, "shape") else BN
    tab10 = jnp.tanh(jnp.tanh(tab[...]) @ eW[...] + eb[...])
    oh_c = (cat[...] == lax.broadcasted_iota(jnp.int32, (BN, 10), 1)).astype(F32)
    classT = _dot_t(tab10, oh_c)
    wtab = jnp.tanh(jnp.tanh(wb[...] @ wW1[...] + wb1[...]) @ wW2[...] + wb2[...])
    oh_b = (bat[...] == lax.broadcasted_iota(jnp.int32, (BN, 64), 1)).astype(F32)
    wallT = _dot_t(wtab, oh_b)
    g1 = jnp.tanh(geo[...] @ gW1[...] + gb1[...])
    geoT = jnp.tanh(_dot_t(gW2c[...], g1) + gb2c[...])
    a1 = jnp.tanh(xt[...] @ iW1[...] + ib1[...])
    h0T = jnp.tanh(_dot_t(iW2c[...], a1) + ib2c[...])
    condT = jnp.concatenate([classT, wallT, geoT], axis=0)
    hT = jnp.concatenate([h0T, condT], axis=0)
    condT_ref[...] = condT
    P_ref[...] = _dot_0(hT, W1d[...]) + b1r[...]
    Q_ref[...] = _dot_0(hT, W1s[...])


_node_prep = pl.pallas_call(
    _node_prep_body,
    grid=(NP // BN,),
    in_specs=[
        pl.BlockSpec((BN, 7), lambda i: (i, 0)),
        pl.BlockSpec((BN, 1), lambda i: (i, 0)),
        pl.BlockSpec((BN, 1), lambda i: (i, 0)),
        pl.BlockSpec((BN, 2), lambda i: (i, 0)),
        pl.BlockSpec((64, 1), lambda i: (0, 0)),
        pl.BlockSpec((7, HID), lambda i: (0, 0)),
        pl.BlockSpec((1, HID), lambda i: (0, 0)),
        pl.BlockSpec((HID, HID), lambda i: (0, 0)),
        pl.BlockSpec((HID, 1), lambda i: (0, 0)),
        pl.BlockSpec((1, EMB), lambda i: (0, 0)),
        pl.BlockSpec((1, EMB), lambda i: (0, 0)),
        pl.BlockSpec((EMB, EMB), lambda i: (0, 0)),
        pl.BlockSpec((1, EMB), lambda i: (0, 0)),
        pl.BlockSpec((10, EMB), lambda i: (0, 0)),
        pl.BlockSpec((EMB, EMB), lambda i: (0, 0)),
        pl.BlockSpec((1, EMB), lambda i: (0, 0)),
        pl.BlockSpec((2, EMB), lambda i: (0, 0)),
        pl.BlockSpec((1, EMB), lambda i: (0, 0)),
        pl.BlockSpec((EMB, EMB), lambda i: (0, 0)),
        pl.BlockSpec((EMB, 1), lambda i: (0, 0)),
        pl.BlockSpec((H2, HID), lambda i: (0, 0)),
        pl.BlockSpec((H2, HID), lambda i: (0, 0)),
        pl.BlockSpec((1, HID), lambda i: (0, 0)),
    ],
    out_specs=[
        pl.BlockSpec((COND, BN), lambda i: (0, i)),
        pl.BlockSpec((BN, HID), lambda i: (i, 0)),
        pl.BlockSpec((BN, HID), lambda i: (i, 0)),
    ],
    out_shape=[
        jax.ShapeDtypeStruct((COND, NP), F32),
        jax.ShapeDtypeStruct((NP, HID), F32),
        jax.ShapeDtypeStruct((NP, HID), F32),
    ],
)


_sc_mesh = plsc.VectorSubcoreMesh(core_axis_name="c", subcore_axis_name="s")


@functools.partial(
    pl.kernel,
    out_type=(jax.ShapeDtypeStruct((EP, HID), F32),
              jax.ShapeDtypeStruct((EP, HID), F32)),
    mesh=_sc_mesh,
    scratch_types=[
        pltpu.VMEM((2, CH_B), jnp.int32),
        pltpu.VMEM((2, CH_B), jnp.int32),
        pltpu.VMEM((2, CH_B, HID), F32),
        pltpu.VMEM((2, CH_B, HID), F32),
        [pltpu.SemaphoreType.DMA] * 2,
        [pltpu.SemaphoreType.DMA] * 2,
        [pltpu.SemaphoreType.DMA] * 2,
        [pltpu.SemaphoreType.DMA] * 2,
    ],
)
def _edge_gather(P_hbm, Q_hbm, dst_hbm, src_hbm, preD_hbm, preS_hbm,
                 dsti, srci, bufP, bufQ, gP, gQ, sP, sQ):
    wid = lax.axis_index("s") * 2 + lax.axis_index("c")
    base = wid * EPW

    def issue(c, b):
        off = base + c * CH_B
        pltpu.sync_copy(dst_hbm.at[pl.ds(off, CH_B)], dsti.at[b])
        pltpu.sync_copy(src_hbm.at[pl.ds(off, CH_B)], srci.at[b])
        pltpu.async_copy(P_hbm.at[dsti.at[b]], bufP.at[b], gP[b])
        pltpu.async_copy(Q_hbm.at[srci.at[b]], bufQ.at[b], gQ[b])

    def drain(c, b):
        off = base + c * CH_B
        pltpu.make_async_copy(P_hbm.at[dsti.at[b]], bufP.at[b], gP[b]).wait()
        pltpu.make_async_copy(Q_hbm.at[srci.at[b]], bufQ.at[b], gQ[b]).wait()
        pltpu.async_copy(bufP.at[b], preD_hbm.at[pl.ds(off, CH_B)], sP[b])
        pltpu.async_copy(bufQ.at[b], preS_hbm.at[pl.ds(off, CH_B)], sQ[b])

    def wstore(c, b):
        off = base + c * CH_B
        pltpu.make_async_copy(bufP.at[b], preD_hbm.at[pl.ds(off, CH_B)], sP[b]).wait()
        pltpu.make_async_copy(bufQ.at[b], preS_hbm.at[pl.ds(off, CH_B)], sQ[b]).wait()

    for c in range(NCH_B):
        b = c % 2
        if c >= 2:
            wstore(c - 2, b)
        issue(c, b)
        if c >= 1:
            drain(c - 1, 1 - b)
    drain(NCH_B - 1, (NCH_B - 1) % 2)
    wstore(NCH_B - 2, (NCH_B - 2) % 2)
    wstore(NCH_B - 1, (NCH_B - 1) % 2)


def _edge_mlp_body(pD, pS, dstp, W2, b2r, eye, out_ref):
    t = jnp.tanh(pD[...] + pS[...])
    M = t @ W2[...] + b2r[...]
    Mr = M.reshape(BE // 2, 2, HID)
    deq = dstp[:, 0:1] == dstp[:, 1:2]
    mm = jnp.max(Mr, axis=1)
    me = jnp.where(deq, mm, Mr[:, 0, :])
    mo = jnp.where(deq, mm, Mr[:, 1, :])
    M2 = jnp.concatenate([me[:, None, :], mo[:, None, :]], axis=1).reshape(BE, HID)
    out_ref[...] = _dot_t(eye[...], M2)


_edge_mlp = pl.pallas_call(
    _edge_mlp_body,
    grid=(EP // BE,),
    in_specs=[
        pl.BlockSpec((BE, HID), lambda i: (i, 0)),
        pl.BlockSpec((BE, HID), lambda i: (i, 0)),
        pl.BlockSpec((BE // 2, 2), lambda i: (i, 0)),
        pl.BlockSpec((HID, HID), lambda i: (0, 0)),
        pl.BlockSpec((1, HID), lambda i: (0, 0)),
        pl.BlockSpec((HID, HID), lambda i: (0, 0)),
    ],
    out_specs=pl.BlockSpec((HID, BE), lambda i: (0, i)),
    out_shape=jax.ShapeDtypeStruct((HID, EP), F32),
)


@functools.partial(
    pl.kernel,
    out_type=jax.ShapeDtypeStruct((EQ, HID, NP), F32),
    mesh=_sc_mesh,
    compiler_params=pltpu.CompilerParams(needs_layout_passes=False),
    scratch_types=[
        pltpu.VMEM((CH_D,), jnp.int32),
        pltpu.VMEM((8, CH_D), F32),
        pltpu.VMEM((8, NP), F32),
    ],
)
def _seg_max(MT_hbm, dst_hbm, out_hbm, dstv, mbuf, acc):
    wid = lax.axis_index("s") * 2 + lax.axis_index("c")
    cg = wid % 16
    eq = wid // 16
    lane = lax.iota(jnp.int32, 16)
    rowp = lane % 8
    colp = lane // 8
    negv = jnp.full((16,), NEG, F32)

    for r in range(8):
        def initrow(c, carry, r=r):
            acc[r, pl.ds(c * 16, 16)] = negv
            return carry
        lax.fori_loop(0, NP // 16, initrow, 0)

    def chunk_step(ci, carry):
        e0 = eq * EPH + ci * CH_D
        pltpu.sync_copy(dst_hbm.at[pl.ds(e0, CH_D)], dstv)
        pltpu.sync_copy(MT_hbm.at[pl.ds(cg * 8, 8), pl.ds(e0, CH_D)], mbuf)

        def pair_step(k, c2):
            j0 = k * (2 * UNROLL)
            dcols = []
            mvs = []
            for u in range(UNROLL):
                j = j0 + 2 * u
                dcols.append(plsc.load_gather(dstv, [j + colp]))
                mvs.append(plsc.load_gather(mbuf, [rowp, j + colp]))
            for u in range(UNROLL):
                cur = plsc.load_gather(acc, [rowp, dcols[u]])
                plsc.store_scatter(acc, [rowp, dcols[u]], jnp.maximum(cur, mvs[u]))
            return c2

        lax.fori_loop(0, PAIRS // UNROLL, pair_step, 0)
        return carry

    lax.fori_loop(0, EPH // CH_D, chunk_step, 0)
    pltpu.sync_copy(acc, out_hbm.at[eq, pl.ds(cg * 8, 8), :])


def _merge_h(s_blk):
    sm = jnp.max(s_blk, axis=0)
    sm = jnp.where(sm < -1.0e38, 0.0, sm)
    return jnp.tanh(sm)


def _mid_body(s, condT, W1d, W1s, b1r, P_ref, Q_ref):
    hT = jnp.concatenate([_merge_h(s[...]), condT[...]], axis=0)
    P_ref[...] = _dot_0(hT, W1d[...]) + b1r[...]
    Q_ref[...] = _dot_0(hT, W1s[...])


_mid = pl.pallas_call(
    _mid_body,
    grid=(NP // BN,),
    in_specs=[
        pl.BlockSpec((EQ, HID, BN), lambda i: (0, 0, i)),
        pl.BlockSpec((COND, BN), lambda i: (0, i)),
        pl.BlockSpec((H2, HID), lambda i: (0, 0)),
        pl.BlockSpec((H2, HID), lambda i: (0, 0)),
        pl.BlockSpec((1, HID), lambda i: (0, 0)),
    ],
    out_specs=[
        pl.BlockSpec((BN, HID), lambda i: (i, 0)),
        pl.BlockSpec((BN, HID), lambda i: (i, 0)),
    ],
    out_shape=[
        jax.ShapeDtypeStruct((NP, HID), F32),
        jax.ShapeDtypeStruct((NP, HID), F32),
    ],
)


def _tail_body(s, condT, tW1, tb1, tW2, tb2, mu_ref, std_ref):
    hT = jnp.concatenate([_merge_h(s[...]), condT[...]], axis=0)
    t = jnp.tanh(_dot_0(hT, tW1[...]) + tb1[...])
    o = t @ tW2[...] + tb2[...]
    mu_ref[...] = jnp.tanh(o[:, 0:3])
    ls = jnp.tanh(o[:, 3:6])
    std_ref[...] = jnp.exp(-5.0 + 3.5 * (ls + 1.0))


_tail = pl.pallas_call(
    _tail_body,
    grid=(NP // BN,),
    in_specs=[
        pl.BlockSpec((EQ, HID, BN), lambda i: (0, 0, i)),
        pl.BlockSpec((COND, BN), lambda i: (0, i)),
        pl.BlockSpec((H2, HID), lambda i: (0, 0)),
        pl.BlockSpec((1, HID), lambda i: (0, 0)),
        pl.BlockSpec((HID, 6), lambda i: (0, 0)),
        pl.BlockSpec((1, 6), lambda i: (0, 0)),
    ],
    out_specs=[
        pl.BlockSpec((BN, 3), lambda i: (i, 0)),
        pl.BlockSpec((BN, 3), lambda i: (i, 0)),
    ],
    out_shape=[
        jax.ShapeDtypeStruct((NP, 3), F32),
        jax.ShapeDtypeStruct((NP, 3), F32),
    ],
)


def kernel(wall_batch, x, tar_scores, geo,
           init_W1, init_b1, init_W2, init_b2,
           wall_W1, wall_b1, wall_W2, wall_b2,
           emb_table, emb_W, emb_b,
           geo_W1, geo_b1, geo_W2, geo_b2,
           c1_W1, c1_b1, c1_W2, c1_b2,
           c2_W1, c2_b1, c2_W2, c2_b2,
           tail_W1, tail_b1, tail_W2, tail_b2,
           category, batch, edge_index):
    zn = NP - N
    xt = jnp.concatenate([x, tar_scores], axis=1)
    xt = jnp.concatenate([xt, jnp.zeros((zn, 7), F32)], axis=0)
    cat_p = jnp.concatenate([category, jnp.zeros((zn, 1), jnp.int32)], axis=0)
    bat2 = jnp.concatenate([batch.reshape(N, 1), jnp.zeros((zn, 1), jnp.int32)], axis=0)
    geo_p = jnp.concatenate([geo, jnp.zeros((zn, 2), F32)], axis=0)
    pad_d = jnp.full((EP - E,), N, jnp.int32)
    pad_s = jnp.zeros((EP - E,), jnp.int32)
    dst = jnp.concatenate([edge_index[1], pad_d])
    src = jnp.concatenate([edge_index[0], pad_s])
    W1d1, W1s1 = c1_W1[:H2] - c1_W1[H2:], c1_W1[H2:]
    W1d2, W1s2 = c2_W1[:H2] - c2_W1[H2:], c2_W1[H2:]

    condT, P1, Q1 = _node_prep(
        xt, cat_p, bat2, geo_p, wall_batch,
        init_W1, init_b1.reshape(1, HID), init_W2, init_b2.reshape(HID, 1),
        wall_W1.reshape(1, EMB), wall_b1.reshape(1, EMB),
        wall_W2, wall_b2.reshape(1, EMB),
        emb_table, emb_W, emb_b.reshape(1, EMB),
        geo_W1, geo_b1.reshape(1, EMB), geo_W2, geo_b2.reshape(EMB, 1),
        W1d1, W1s1, c1_b1.reshape(1, HID))

    dstp = dst.reshape(EP // 2, 2)
    eye = jnp.eye(HID, dtype=F32)
    preD, preS = _edge_gather(P1, Q1, dst, src)
    MT1 = _edge_mlp(preD, preS, dstp, c1_W2, c1_b2.reshape(1, HID), eye)
    S1 = _seg_max(MT1, dst)

    P2, Q2 = _mid(S1, condT, W1d2, W1s2, c2_b1.reshape(1, HID))
    preD2, preS2 = _edge_gather(P2, Q2, dst, src)
    MT2 = _edge_mlp(preD2, preS2, dstp, c2_W2, c2_b2.reshape(1, HID), eye)
    S2 = _seg_max(MT2, dst)

    mu, std = _tail(S2, condT, tail_W1, tail_b1.reshape(1, HID),
                    tail_W2, tail_b2.reshape(1, 6))
    return (mu[:N], std[:N])


# speedup vs baseline: 1.7568x; 1.1819x over previous
import functools

import jax
import jax.numpy as jnp
from jax import lax
from jax.experimental import pallas as pl
from jax.experimental.pallas import tpu as pltpu
from jax.experimental.pallas import tpu_sc as plsc

N = 10000
NP = 10240
E = 160000
EP = 163840
HID = 128
EMB = 64
COND = 3 * EMB
H2 = HID + COND
BN = 2048
BE = 4096
NW = 32
EPH2 = EP // 2
EPW = EPH2 // NW
CH_B = 128
NCH_B = EPW // CH_B
EQ = 2
SH = EPH2 // EQ
CH_D = 5120
PAIRS = CH_D // 2
UNROLL = 4
NEG = -3.0e38
F32 = jnp.float32

_dn_t = (((0,), (1,)), ((), ()))
_dn_0 = (((0,), (0,)), ((), ()))


def _dot_t(a, b):
    return lax.dot_general(a, b, _dn_t, preferred_element_type=F32)


def _dot_0(a, b):
    return lax.dot_general(a, b, _dn_0, preferred_element_type=F32)


def _node_prep_body(xt, cat, bat, geo, wb,
                    iW1, ib1, iW2c, ib2c,
                    wW1, wb1, wW2, wb2,
                    tab, eW, eb,
                    gW1, gb1, gW2c, gb2c,
                    W1d, W1s, b1r,
                    condT_ref, P_ref, Q_ref):
    bn = xt.shape[0] if hasattr(xt, "shape") else BN
    tab10 = jnp.tanh(jnp.tanh(tab[...]) @ eW[...] + eb[...])
    oh_c = (cat[...] == lax.broadcasted_iota(jnp.int32, (BN, 10), 1)).astype(F32)
    classT = _dot_t(tab10, oh_c)
    wtab = jnp.tanh(jnp.tanh(wb[...] @ wW1[...] + wb1[...]) @ wW2[...] + wb2[...])
    oh_b = (bat[...] == lax.broadcasted_iota(jnp.int32, (BN, 64), 1)).astype(F32)
    wallT = _dot_t(wtab, oh_b)
    g1 = jnp.tanh(geo[...] @ gW1[...] + gb1[...])
    geoT = jnp.tanh(_dot_t(gW2c[...], g1) + gb2c[...])
    a1 = jnp.tanh(xt[...] @ iW1[...] + ib1[...])
    h0T = jnp.tanh(_dot_t(iW2c[...], a1) + ib2c[...])
    condT = jnp.concatenate([classT, wallT, geoT], axis=0)
    hT = jnp.concatenate([h0T, condT], axis=0)
    condT_ref[...] = condT
    P_ref[...] = _dot_0(hT, W1d[...]) + b1r[...]
    Q_ref[...] = _dot_0(hT, W1s[...])


_node_prep = pl.pallas_call(
    _node_prep_body,
    grid=(NP // BN,),
    in_specs=[
        pl.BlockSpec((BN, 7), lambda i: (i, 0)),
        pl.BlockSpec((BN, 1), lambda i: (i, 0)),
        pl.BlockSpec((BN, 1), lambda i: (i, 0)),
        pl.BlockSpec((BN, 2), lambda i: (i, 0)),
        pl.BlockSpec((64, 1), lambda i: (0, 0)),
        pl.BlockSpec((7, HID), lambda i: (0, 0)),
        pl.BlockSpec((1, HID), lambda i: (0, 0)),
        pl.BlockSpec((HID, HID), lambda i: (0, 0)),
        pl.BlockSpec((HID, 1), lambda i: (0, 0)),
        pl.BlockSpec((1, EMB), lambda i: (0, 0)),
        pl.BlockSpec((1, EMB), lambda i: (0, 0)),
        pl.BlockSpec((EMB, EMB), lambda i: (0, 0)),
        pl.BlockSpec((1, EMB), lambda i: (0, 0)),
        pl.BlockSpec((10, EMB), lambda i: (0, 0)),
        pl.BlockSpec((EMB, EMB), lambda i: (0, 0)),
        pl.BlockSpec((1, EMB), lambda i: (0, 0)),
        pl.BlockSpec((2, EMB), lambda i: (0, 0)),
        pl.BlockSpec((1, EMB), lambda i: (0, 0)),
        pl.BlockSpec((EMB, EMB), lambda i: (0, 0)),
        pl.BlockSpec((EMB, 1), lambda i: (0, 0)),
        pl.BlockSpec((H2, HID), lambda i: (0, 0)),
        pl.BlockSpec((H2, HID), lambda i: (0, 0)),
        pl.BlockSpec((1, HID), lambda i: (0, 0)),
    ],
    out_specs=[
        pl.BlockSpec((COND, BN), lambda i: (0, i)),
        pl.BlockSpec((BN, HID), lambda i: (i, 0)),
        pl.BlockSpec((BN, HID), lambda i: (i, 0)),
    ],
    out_shape=[
        jax.ShapeDtypeStruct((COND, NP), F32),
        jax.ShapeDtypeStruct((NP, HID), F32),
        jax.ShapeDtypeStruct((NP, HID), F32),
    ],
)


_sc_mesh = plsc.VectorSubcoreMesh(core_axis_name="c", subcore_axis_name="s")


@functools.partial(
    pl.kernel,
    out_type=(jax.ShapeDtypeStruct((EPH2, HID), F32),
              jax.ShapeDtypeStruct((EPH2, HID), F32)),
    mesh=_sc_mesh,
    scratch_types=[
        pltpu.VMEM((2, CH_B), jnp.int32),
        pltpu.VMEM((2, CH_B), jnp.int32),
        pltpu.VMEM((2, CH_B, HID), F32),
        pltpu.VMEM((2, CH_B, HID), F32),
        [pltpu.SemaphoreType.DMA] * 2,
        [pltpu.SemaphoreType.DMA] * 2,
        [pltpu.SemaphoreType.DMA] * 2,
        [pltpu.SemaphoreType.DMA] * 2,
    ],
)
def _edge_gather(P_hbm, Q_hbm, dst_hbm, src_hbm, preD_hbm, preS_hbm,
                 dsti, srci, bufP, bufQ, gP, gQ, sP, sQ):
    wid = lax.axis_index("s") * 2 + lax.axis_index("c")
    base = wid * EPW

    def issue(c, b):
        off = base + c * CH_B
        pltpu.sync_copy(dst_hbm.at[pl.ds(off, CH_B)], dsti.at[b])
        pltpu.sync_copy(src_hbm.at[pl.ds(off, CH_B)], srci.at[b])
        pltpu.async_copy(P_hbm.at[dsti.at[b]], bufP.at[b], gP[b])
        pltpu.async_copy(Q_hbm.at[srci.at[b]], bufQ.at[b], gQ[b])

    def drain(c, b):
        off = base + c * CH_B
        pltpu.make_async_copy(P_hbm.at[dsti.at[b]], bufP.at[b], gP[b]).wait()
        pltpu.make_async_copy(Q_hbm.at[srci.at[b]], bufQ.at[b], gQ[b]).wait()
        pltpu.async_copy(bufP.at[b], preD_hbm.at[pl.ds(off, CH_B)], sP[b])
        pltpu.async_copy(bufQ.at[b], preS_hbm.at[pl.ds(off, CH_B)], sQ[b])

    def wstore(c, b):
        off = base + c * CH_B
        pltpu.make_async_copy(bufP.at[b], preD_hbm.at[pl.ds(off, CH_B)], sP[b]).wait()
        pltpu.make_async_copy(bufQ.at[b], preS_hbm.at[pl.ds(off, CH_B)], sQ[b]).wait()

    for c in range(NCH_B):
        b = c % 2
        if c >= 2:
            wstore(c - 2, b)
        issue(c, b)
        if c >= 1:
            drain(c - 1, 1 - b)
    drain(NCH_B - 1, (NCH_B - 1) % 2)
    wstore(NCH_B - 2, (NCH_B - 2) % 2)
    wstore(NCH_B - 1, (NCH_B - 1) % 2)


def _edge_mlp_body(pD, pS, dstp, W2, b2r, eye, out_ref):
    t = jnp.tanh(pD[...] + pS[...])
    M = t @ W2[...] + b2r[...]
    Mr = M.reshape(BE // 2, 2, HID)
    deq = dstp[:, 0:1] == dstp[:, 1:2]
    mm = jnp.max(Mr, axis=1)
    me = jnp.where(deq, mm, Mr[:, 0, :])
    mo = jnp.where(deq, mm, Mr[:, 1, :])
    M2 = jnp.concatenate([me[:, None, :], mo[:, None, :]], axis=1).reshape(BE, HID)
    out_ref[...] = _dot_t(eye[...], M2)


_edge_mlp = pl.pallas_call(
    _edge_mlp_body,
    grid=(EPH2 // BE,),
    in_specs=[
        pl.BlockSpec((BE, HID), lambda i: (i, 0)),
        pl.BlockSpec((BE, HID), lambda i: (i, 0)),
        pl.BlockSpec((BE // 2, 2), lambda i: (i, 0)),
        pl.BlockSpec((HID, HID), lambda i: (0, 0)),
        pl.BlockSpec((1, HID), lambda i: (0, 0)),
        pl.BlockSpec((HID, HID), lambda i: (0, 0)),
    ],
    out_specs=pl.BlockSpec((HID, BE), lambda i: (0, i)),
    out_shape=jax.ShapeDtypeStruct((HID, EPH2), F32),
)


@functools.partial(
    pl.kernel,
    out_type=jax.ShapeDtypeStruct((EQ, HID, NP), F32),
    mesh=_sc_mesh,
    compiler_params=pltpu.CompilerParams(needs_layout_passes=False),
    scratch_types=[
        pltpu.VMEM((CH_D,), jnp.int32),
        pltpu.VMEM((8, CH_D), F32),
        pltpu.VMEM((8, NP), F32),
    ],
)
def _seg_max(MT_hbm, dst_hbm, out_hbm, dstv, mbuf, acc):
    wid = lax.axis_index("s") * 2 + lax.axis_index("c")
    cg = wid % 16
    eq = wid // 16
    lane = lax.iota(jnp.int32, 16)
    rowp = lane % 8
    colp = lane // 8
    negv = jnp.full((16,), NEG, F32)

    for r in range(8):
        def initrow(c, carry, r=r):
            acc[r, pl.ds(c * 16, 16)] = negv
            return carry
        lax.fori_loop(0, NP // 16, initrow, 0)

    def chunk_step(ci, carry):
        e0 = eq * SH + ci * CH_D
        pltpu.sync_copy(dst_hbm.at[pl.ds(e0, CH_D)], dstv)
        pltpu.sync_copy(MT_hbm.at[pl.ds(cg * 8, 8), pl.ds(e0, CH_D)], mbuf)

        def pair_step(k, c2):
            j0 = k * (2 * UNROLL)
            dcols = []
            mvs = []
            for u in range(UNROLL):
                j = j0 + 2 * u
                dcols.append(plsc.load_gather(dstv, [j + colp]))
                mvs.append(plsc.load_gather(mbuf, [rowp, j + colp]))
            for u in range(UNROLL):
                cur = plsc.load_gather(acc, [rowp, dcols[u]])
                plsc.store_scatter(acc, [rowp, dcols[u]], jnp.maximum(cur, mvs[u]))
            return c2

        lax.fori_loop(0, PAIRS // UNROLL, pair_step, 0)
        return carry

    lax.fori_loop(0, SH // CH_D, chunk_step, 0)
    pltpu.sync_copy(acc, out_hbm.at[eq, pl.ds(cg * 8, 8), :])


def _merge_h(s_blk):
    sm = jnp.max(s_blk, axis=0)
    sm = jnp.where(sm < -1.0e38, 0.0, sm)
    return jnp.tanh(sm)


def _mid_body(sa, sb, condT, W1d, W1s, b1r, P_ref, Q_ref):
    s2 = jnp.maximum(sa[...], sb[...])
    hT = jnp.concatenate([_merge_h(s2), condT[...]], axis=0)
    P_ref[...] = _dot_0(hT, W1d[...]) + b1r[...]
    Q_ref[...] = _dot_0(hT, W1s[...])


_mid = pl.pallas_call(
    _mid_body,
    grid=(NP // BN,),
    in_specs=[
        pl.BlockSpec((EQ, HID, BN), lambda i: (0, 0, i)),
        pl.BlockSpec((EQ, HID, BN), lambda i: (0, 0, i)),
        pl.BlockSpec((COND, BN), lambda i: (0, i)),
        pl.BlockSpec((H2, HID), lambda i: (0, 0)),
        pl.BlockSpec((H2, HID), lambda i: (0, 0)),
        pl.BlockSpec((1, HID), lambda i: (0, 0)),
    ],
    out_specs=[
        pl.BlockSpec((BN, HID), lambda i: (i, 0)),
        pl.BlockSpec((BN, HID), lambda i: (i, 0)),
    ],
    out_shape=[
        jax.ShapeDtypeStruct((NP, HID), F32),
        jax.ShapeDtypeStruct((NP, HID), F32),
    ],
)


def _tail_body(sa, sb, condT, tW1, tb1, tW2, tb2, mu_ref, std_ref):
    s2 = jnp.maximum(sa[...], sb[...])
    hT = jnp.concatenate([_merge_h(s2), condT[...]], axis=0)
    t = jnp.tanh(_dot_0(hT, tW1[...]) + tb1[...])
    o = t @ tW2[...] + tb2[...]
    mu_ref[...] = jnp.tanh(o[:, 0:3])
    ls = jnp.tanh(o[:, 3:6])
    std_ref[...] = jnp.exp(-5.0 + 3.5 * (ls + 1.0))


_tail = pl.pallas_call(
    _tail_body,
    grid=(NP // BN,),
    in_specs=[
        pl.BlockSpec((EQ, HID, BN), lambda i: (0, 0, i)),
        pl.BlockSpec((EQ, HID, BN), lambda i: (0, 0, i)),
        pl.BlockSpec((COND, BN), lambda i: (0, i)),
        pl.BlockSpec((H2, HID), lambda i: (0, 0)),
        pl.BlockSpec((1, HID), lambda i: (0, 0)),
        pl.BlockSpec((HID, 6), lambda i: (0, 0)),
        pl.BlockSpec((1, 6), lambda i: (0, 0)),
    ],
    out_specs=[
        pl.BlockSpec((BN, 3), lambda i: (i, 0)),
        pl.BlockSpec((BN, 3), lambda i: (i, 0)),
    ],
    out_shape=[
        jax.ShapeDtypeStruct((NP, 3), F32),
        jax.ShapeDtypeStruct((NP, 3), F32),
    ],
)


def kernel(wall_batch, x, tar_scores, geo,
           init_W1, init_b1, init_W2, init_b2,
           wall_W1, wall_b1, wall_W2, wall_b2,
           emb_table, emb_W, emb_b,
           geo_W1, geo_b1, geo_W2, geo_b2,
           c1_W1, c1_b1, c1_W2, c1_b2,
           c2_W1, c2_b1, c2_W2, c2_b2,
           tail_W1, tail_b1, tail_W2, tail_b2,
           category, batch, edge_index):
    zn = NP - N
    xt = jnp.concatenate([x, tar_scores], axis=1)
    xt = jnp.concatenate([xt, jnp.zeros((zn, 7), F32)], axis=0)
    cat_p = jnp.concatenate([category, jnp.zeros((zn, 1), jnp.int32)], axis=0)
    bat2 = jnp.concatenate([batch.reshape(N, 1), jnp.zeros((zn, 1), jnp.int32)], axis=0)
    geo_p = jnp.concatenate([geo, jnp.zeros((zn, 2), F32)], axis=0)
    pad_d = jnp.full((EP - E,), N, jnp.int32)
    pad_s = jnp.zeros((EP - E,), jnp.int32)
    dst = jnp.concatenate([edge_index[1], pad_d])
    src = jnp.concatenate([edge_index[0], pad_s])
    W1d1, W1s1 = c1_W1[:H2] - c1_W1[H2:], c1_W1[H2:]
    W1d2, W1s2 = c2_W1[:H2] - c2_W1[H2:], c2_W1[H2:]

    condT, P1, Q1 = _node_prep(
        xt, cat_p, bat2, geo_p, wall_batch,
        init_W1, init_b1.reshape(1, HID), init_W2, init_b2.reshape(HID, 1),
        wall_W1.reshape(1, EMB), wall_b1.reshape(1, EMB),
        wall_W2, wall_b2.reshape(1, EMB),
        emb_table, emb_W, emb_b.reshape(1, EMB),
        geo_W1, geo_b1.reshape(1, EMB), geo_W2, geo_b2.reshape(EMB, 1),
        W1d1, W1s1, c1_b1.reshape(1, HID))

    eye = jnp.eye(HID, dtype=F32)
    dst0, dst1 = dst[:EPH2], dst[EPH2:]
    src0, src1 = src[:EPH2], src[EPH2:]
    dstp0 = dst0.reshape(EPH2 // 2, 2)
    dstp1 = dst1.reshape(EPH2 // 2, 2)

    def conv(P, Q, W2, b2):
        pda, psa = _edge_gather(P, Q, dst0, src0)
        pdb, psb = _edge_gather(P, Q, dst1, src1)
        MTa = _edge_mlp(pda, psa, dstp0, W2, b2.reshape(1, HID), eye)
        MTb = _edge_mlp(pdb, psb, dstp1, W2, b2.reshape(1, HID), eye)
        Sa = _seg_max(MTa, dst0)
        Sb = _seg_max(MTb, dst1)
        return Sa, Sb

    S1a, S1b = conv(P1, Q1, c1_W2, c1_b2)
    P2, Q2 = _mid(S1a, S1b, condT, W1d2, W1s2, c2_b1.reshape(1, HID))
    S2a, S2b = conv(P2, Q2, c2_W2, c2_b2)
    mu, std = _tail(S2a, S2b, condT, tail_W1, tail_b1.reshape(1, HID),
                    tail_W2, tail_b2.reshape(1, 6))
    return (mu[:N], std[:N])


# speedup vs baseline: 1.7969x; 1.0228x over previous
import functools

import jax
import jax.numpy as jnp
from jax import lax
from jax.experimental import pallas as pl
from jax.experimental.pallas import tpu as pltpu
from jax.experimental.pallas import tpu_sc as plsc

N = 10000
NP = 10240
E = 160000
EP = 163840
HID = 128
EMB = 64
COND = 3 * EMB
H2 = HID + COND
BN = 2048
BE = 4096
NW = 32
EPH2 = EP // 2
EPW = EPH2 // NW
CH_B = 128
NCH_B = EPW // CH_B
EQ = 2
SH = EPH2 // EQ
CH_D = 5120
PAIRS = CH_D // 2
UNROLL = 4
NEG = -3.0e38
F32 = jnp.float32

_dn_t = (((0,), (1,)), ((), ()))
_dn_0 = (((0,), (0,)), ((), ()))


def _dot_t(a, b):
    return lax.dot_general(a, b, _dn_t, preferred_element_type=F32)


def _dot_0(a, b):
    return lax.dot_general(a, b, _dn_0, preferred_element_type=F32)


def _node_prep_body(xt, cat, bat, geo, wb,
                    iW1, ib1, iW2c, ib2c,
                    wW1, wb1, wW2, wb2,
                    tab, eW, eb,
                    gW1, gb1, gW2c, gb2c,
                    W1d, W1s, b1r,
                    condT_ref, P_ref, Q_ref):
    bn = xt.shape[0] if hasattr(xt, "shape") else BN
    tab10 = jnp.tanh(jnp.tanh(tab[...]) @ eW[...] + eb[...])
    oh_c = (cat[...] == lax.broadcasted_iota(jnp.int32, (BN, 10), 1)).astype(F32)
    classT = _dot_t(tab10, oh_c)
    wtab = jnp.tanh(jnp.tanh(wb[...] @ wW1[...] + wb1[...]) @ wW2[...] + wb2[...])
    oh_b = (bat[...] == lax.broadcasted_iota(jnp.int32, (BN, 64), 1)).astype(F32)
    wallT = _dot_t(wtab, oh_b)
    g1 = jnp.tanh(geo[...] @ gW1[...] + gb1[...])
    geoT = jnp.tanh(_dot_t(gW2c[...], g1) + gb2c[...])
    a1 = jnp.tanh(xt[...] @ iW1[...] + ib1[...])
    h0T = jnp.tanh(_dot_t(iW2c[...], a1) + ib2c[...])
    condT = jnp.concatenate([classT, wallT, geoT], axis=0)
    hT = jnp.concatenate([h0T, condT], axis=0)
    condT_ref[...] = condT
    P_ref[...] = _dot_0(hT, W1d[...]) + b1r[...]
    Q_ref[...] = _dot_0(hT, W1s[...])


_node_prep = pl.pallas_call(
    _node_prep_body,
    grid=(NP // BN,),
    in_specs=[
        pl.BlockSpec((BN, 7), lambda i: (i, 0)),
        pl.BlockSpec((BN, 1), lambda i: (i, 0)),
        pl.BlockSpec((BN, 1), lambda i: (i, 0)),
        pl.BlockSpec((BN, 2), lambda i: (i, 0)),
        pl.BlockSpec((64, 1), lambda i: (0, 0)),
        pl.BlockSpec((7, HID), lambda i: (0, 0)),
        pl.BlockSpec((1, HID), lambda i: (0, 0)),
        pl.BlockSpec((HID, HID), lambda i: (0, 0)),
        pl.BlockSpec((HID, 1), lambda i: (0, 0)),
        pl.BlockSpec((1, EMB), lambda i: (0, 0)),
        pl.BlockSpec((1, EMB), lambda i: (0, 0)),
        pl.BlockSpec((EMB, EMB), lambda i: (0, 0)),
        pl.BlockSpec((1, EMB), lambda i: (0, 0)),
        pl.BlockSpec((10, EMB), lambda i: (0, 0)),
        pl.BlockSpec((EMB, EMB), lambda i: (0, 0)),
        pl.BlockSpec((1, EMB), lambda i: (0, 0)),
        pl.BlockSpec((2, EMB), lambda i: (0, 0)),
        pl.BlockSpec((1, EMB), lambda i: (0, 0)),
        pl.BlockSpec((EMB, EMB), lambda i: (0, 0)),
        pl.BlockSpec((EMB, 1), lambda i: (0, 0)),
        pl.BlockSpec((H2, HID), lambda i: (0, 0)),
        pl.BlockSpec((H2, HID), lambda i: (0, 0)),
        pl.BlockSpec((1, HID), lambda i: (0, 0)),
    ],
    out_specs=[
        pl.BlockSpec((COND, BN), lambda i: (0, i)),
        pl.BlockSpec((BN, HID), lambda i: (i, 0)),
        pl.BlockSpec((BN, HID), lambda i: (i, 0)),
    ],
    out_shape=[
        jax.ShapeDtypeStruct((COND, NP), F32),
        jax.ShapeDtypeStruct((NP, HID), F32),
        jax.ShapeDtypeStruct((NP, HID), F32),
    ],
)


_sc_mesh = plsc.VectorSubcoreMesh(core_axis_name="c", subcore_axis_name="s")


@functools.partial(
    pl.kernel,
    out_type=jax.ShapeDtypeStruct((EPH2, HID), F32),
    mesh=_sc_mesh,
    compiler_params=pltpu.CompilerParams(needs_layout_passes=False),
    scratch_types=[
        pltpu.VMEM((2, CH_B), jnp.int32),
        pltpu.VMEM((2, CH_B), jnp.int32),
        pltpu.VMEM((2, CH_B, HID), F32),
        pltpu.VMEM((2, CH_B, HID), F32),
        [pltpu.SemaphoreType.DMA] * 2,
        [pltpu.SemaphoreType.DMA] * 2,
        [pltpu.SemaphoreType.DMA] * 2,
    ],
)
def _edge_gather(P_hbm, Q_hbm, dst_hbm, src_hbm, pre_hbm,
                 dsti, srci, bufP, bufQ, gP, gQ, sP):
    wid = lax.axis_index("s") * 2 + lax.axis_index("c")
    base = wid * EPW

    def issue(c, b):
        off = base + c * CH_B
        pltpu.sync_copy(dst_hbm.at[pl.ds(off, CH_B)], dsti.at[b])
        pltpu.sync_copy(src_hbm.at[pl.ds(off, CH_B)], srci.at[b])
        pltpu.async_copy(P_hbm.at[dsti.at[b]], bufP.at[b], gP[b])
        pltpu.async_copy(Q_hbm.at[srci.at[b]], bufQ.at[b], gQ[b])

    def drain(c, b):
        off = base + c * CH_B
        pltpu.make_async_copy(P_hbm.at[dsti.at[b]], bufP.at[b], gP[b]).wait()
        pltpu.make_async_copy(Q_hbm.at[srci.at[b]], bufQ.at[b], gQ[b]).wait()

        def addrow(i, carry):
            for k in range(HID // 16):
                sl = pl.ds(k * 16, 16)
                bufP[b, i, sl] = bufP[b, i, sl] + bufQ[b, i, sl]
            return carry

        lax.fori_loop(0, CH_B, addrow, 0)
        pltpu.async_copy(bufP.at[b], pre_hbm.at[pl.ds(off, CH_B)], sP[b])

    def wstore(c, b):
        off = base + c * CH_B
        pltpu.make_async_copy(bufP.at[b], pre_hbm.at[pl.ds(off, CH_B)], sP[b]).wait()

    for c in range(NCH_B):
        b = c % 2
        if c >= 2:
            wstore(c - 2, b)
        issue(c, b)
        if c >= 1:
            drain(c - 1, 1 - b)
    drain(NCH_B - 1, (NCH_B - 1) % 2)
    wstore(NCH_B - 2, (NCH_B - 2) % 2)
    wstore(NCH_B - 1, (NCH_B - 1) % 2)


def _edge_mlp_body(pre, dstp, W2, b2r, eye, out_ref):
    t = jnp.tanh(pre[...])
    M = t @ W2[...] + b2r[...]
    Mr = M.reshape(BE // 2, 2, HID)
    deq = dstp[:, 0:1] == dstp[:, 1:2]
    mm = jnp.max(Mr, axis=1)
    me = jnp.where(deq, mm, Mr[:, 0, :])
    mo = jnp.where(deq, mm, Mr[:, 1, :])
    M2 = jnp.concatenate([me[:, None, :], mo[:, None, :]], axis=1).reshape(BE, HID)
    out_ref[...] = _dot_t(eye[...], M2)


_edge_mlp = pl.pallas_call(
    _edge_mlp_body,
    grid=(EPH2 // BE,),
    in_specs=[
        pl.BlockSpec((BE, HID), lambda i: (i, 0)),
        pl.BlockSpec((BE // 2, 2), lambda i: (i, 0)),
        pl.BlockSpec((HID, HID), lambda i: (0, 0)),
        pl.BlockSpec((1, HID), lambda i: (0, 0)),
        pl.BlockSpec((HID, HID), lambda i: (0, 0)),
    ],
    out_specs=pl.BlockSpec((HID, BE), lambda i: (0, i)),
    out_shape=jax.ShapeDtypeStruct((HID, EPH2), F32),
)


@functools.partial(
    pl.kernel,
    out_type=jax.ShapeDtypeStruct((EQ, HID, NP), F32),
    mesh=_sc_mesh,
    compiler_params=pltpu.CompilerParams(needs_layout_passes=False),
    scratch_types=[
        pltpu.VMEM((CH_D,), jnp.int32),
        pltpu.VMEM((8, CH_D), F32),
        pltpu.VMEM((8, NP), F32),
    ],
)
def _seg_max(MT_hbm, dst_hbm, out_hbm, dstv, mbuf, acc):
    wid = lax.axis_index("s") * 2 + lax.axis_index("c")
    cg = wid % 16
    eq = wid // 16
    lane = lax.iota(jnp.int32, 16)
    rowp = lane % 8
    colp = lane // 8
    negv = jnp.full((16,), NEG, F32)

    for r in range(8):
        def initrow(c, carry, r=r):
            acc[r, pl.ds(c * 16, 16)] = negv
            return carry
        lax.fori_loop(0, NP // 16, initrow, 0)

    def chunk_step(ci, carry):
        e0 = eq * SH + ci * CH_D
        pltpu.sync_copy(dst_hbm.at[pl.ds(e0, CH_D)], dstv)
        pltpu.sync_copy(MT_hbm.at[pl.ds(cg * 8, 8), pl.ds(e0, CH_D)], mbuf)

        def pair_step(k, c2):
            j0 = k * (2 * UNROLL)
            dcols = []
            mvs = []
            for u in range(UNROLL):
                j = j0 + 2 * u
                dcols.append(plsc.load_gather(dstv, [j + colp]))
                mvs.append(plsc.load_gather(mbuf, [rowp, j + colp]))
            for u in range(UNROLL):
                cur = plsc.load_gather(acc, [rowp, dcols[u]])
                plsc.store_scatter(acc, [rowp, dcols[u]], jnp.maximum(cur, mvs[u]))
            return c2

        lax.fori_loop(0, PAIRS // UNROLL, pair_step, 0)
        return carry

    lax.fori_loop(0, SH // CH_D, chunk_step, 0)
    pltpu.sync_copy(acc, out_hbm.at[eq, pl.ds(cg * 8, 8), :])


def _merge_h(s_blk):
    sm = jnp.max(s_blk, axis=0)
    sm = jnp.where(sm < -1.0e38, 0.0, sm)
    return jnp.tanh(sm)


def _mid_body(sa, sb, condT, W1d, W1s, b1r, P_ref, Q_ref):
    s2 = jnp.maximum(sa[...], sb[...])
    hT = jnp.concatenate([_merge_h(s2), condT[...]], axis=0)
    P_ref[...] = _dot_0(hT, W1d[...]) + b1r[...]
    Q_ref[...] = _dot_0(hT, W1s[...])


_mid = pl.pallas_call(
    _mid_body,
    grid=(NP // BN,),
    in_specs=[
        pl.BlockSpec((EQ, HID, BN), lambda i: (0, 0, i)),
        pl.BlockSpec((EQ, HID, BN), lambda i: (0, 0, i)),
        pl.BlockSpec((COND, BN), lambda i: (0, i)),
        pl.BlockSpec((H2, HID), lambda i: (0, 0)),
        pl.BlockSpec((H2, HID), lambda i: (0, 0)),
        pl.BlockSpec((1, HID), lambda i: (0, 0)),
    ],
    out_specs=[
        pl.BlockSpec((BN, HID), lambda i: (i, 0)),
        pl.BlockSpec((BN, HID), lambda i: (i, 0)),
    ],
    out_shape=[
        jax.ShapeDtypeStruct((NP, HID), F32),
        jax.ShapeDtypeStruct((NP, HID), F32),
    ],
)


def _tail_body(sa, sb, condT, tW1, tb1, tW2, tb2, mu_ref, std_ref):
    s2 = jnp.maximum(sa[...], sb[...])
    hT = jnp.concatenate([_merge_h(s2), condT[...]], axis=0)
    t = jnp.tanh(_dot_0(hT, tW1[...]) + tb1[...])
    o = t @ tW2[...] + tb2[...]
    mu_ref[...] = jnp.tanh(o[:, 0:3])
    ls = jnp.tanh(o[:, 3:6])
    std_ref[...] = jnp.exp(-5.0 + 3.5 * (ls + 1.0))


_tail = pl.pallas_call(
    _tail_body,
    grid=(NP // BN,),
    in_specs=[
        pl.BlockSpec((EQ, HID, BN), lambda i: (0, 0, i)),
        pl.BlockSpec((EQ, HID, BN), lambda i: (0, 0, i)),
        pl.BlockSpec((COND, BN), lambda i: (0, i)),
        pl.BlockSpec((H2, HID), lambda i: (0, 0)),
        pl.BlockSpec((1, HID), lambda i: (0, 0)),
        pl.BlockSpec((HID, 6), lambda i: (0, 0)),
        pl.BlockSpec((1, 6), lambda i: (0, 0)),
    ],
    out_specs=[
        pl.BlockSpec((BN, 3), lambda i: (i, 0)),
        pl.BlockSpec((BN, 3), lambda i: (i, 0)),
    ],
    out_shape=[
        jax.ShapeDtypeStruct((NP, 3), F32),
        jax.ShapeDtypeStruct((NP, 3), F32),
    ],
)


def kernel(wall_batch, x, tar_scores, geo,
           init_W1, init_b1, init_W2, init_b2,
           wall_W1, wall_b1, wall_W2, wall_b2,
           emb_table, emb_W, emb_b,
           geo_W1, geo_b1, geo_W2, geo_b2,
           c1_W1, c1_b1, c1_W2, c1_b2,
           c2_W1, c2_b1, c2_W2, c2_b2,
           tail_W1, tail_b1, tail_W2, tail_b2,
           category, batch, edge_index):
    zn = NP - N
    xt = jnp.concatenate([x, tar_scores], axis=1)
    xt = jnp.concatenate([xt, jnp.zeros((zn, 7), F32)], axis=0)
    cat_p = jnp.concatenate([category, jnp.zeros((zn, 1), jnp.int32)], axis=0)
    bat2 = jnp.concatenate([batch.reshape(N, 1), jnp.zeros((zn, 1), jnp.int32)], axis=0)
    geo_p = jnp.concatenate([geo, jnp.zeros((zn, 2), F32)], axis=0)
    pad_d = jnp.full((EP - E,), N, jnp.int32)
    pad_s = jnp.zeros((EP - E,), jnp.int32)
    dst = jnp.concatenate([edge_index[1], pad_d])
    src = jnp.concatenate([edge_index[0], pad_s])
    W1d1, W1s1 = c1_W1[:H2] - c1_W1[H2:], c1_W1[H2:]
    W1d2, W1s2 = c2_W1[:H2] - c2_W1[H2:], c2_W1[H2:]

    condT, P1, Q1 = _node_prep(
        xt, cat_p, bat2, geo_p, wall_batch,
        init_W1, init_b1.reshape(1, HID), init_W2, init_b2.reshape(HID, 1),
        wall_W1.reshape(1, EMB), wall_b1.reshape(1, EMB),
        wall_W2, wall_b2.reshape(1, EMB),
        emb_table, emb_W, emb_b.reshape(1, EMB),
        geo_W1, geo_b1.reshape(1, EMB), geo_W2, geo_b2.reshape(EMB, 1),
        W1d1, W1s1, c1_b1.reshape(1, HID))

    eye = jnp.eye(HID, dtype=F32)
    dst0, dst1 = dst[:EPH2], dst[EPH2:]
    src0, src1 = src[:EPH2], src[EPH2:]
    dstp0 = dst0.reshape(EPH2 // 2, 2)
    dstp1 = dst1.reshape(EPH2 // 2, 2)

    def conv(P, Q, W2, b2):
        pra = _edge_gather(P, Q, dst0, src0)
        prb = _edge_gather(P, Q, dst1, src1)
        MTa = _edge_mlp(pra, dstp0, W2, b2.reshape(1, HID), eye)
        MTb = _edge_mlp(prb, dstp1, W2, b2.reshape(1, HID), eye)
        Sa = _seg_max(MTa, dst0)
        Sb = _seg_max(MTb, dst1)
        return Sa, Sb

    S1a, S1b = conv(P1, Q1, c1_W2, c1_b2)
    P2, Q2 = _mid(S1a, S1b, condT, W1d2, W1s2, c2_b1.reshape(1, HID))
    S2a, S2b = conv(P2, Q2, c2_W2, c2_b2)
    mu, std = _tail(S2a, S2b, condT, tail_W1, tail_b1.reshape(1, HID),
                    tail_W2, tail_b2.reshape(1, 6))
    return (mu[:N], std[:N])
